# Initial kernel scaffold; baseline (speedup 1.0000x reference)
import jax
import jax.numpy as jnp
from jax.experimental import pallas as pl


def kernel(memory, commit_bits, commit_vals, query_bits, bit_sel):
    raise NotImplementedError("write your pallas kernel here")


# speedup vs baseline: 40.2718x; 40.2718x over previous
import functools

import jax
import jax.numpy as jnp
from jax import lax
from jax.experimental import pallas as pl
from jax.experimental.pallas import tpu as pltpu
from jax.experimental.pallas import tpu_sc as plsc

_NC = 2
_NS = 16
_NW = _NC * _NS
_L = 16

_SENT = -1.0


def _addr_body(cb_ref, qb_ref, w_ref, cf_ref, qf_ref, *, ncells):
  w = w_ref[...]
  n = w.shape[1]
  offs = lax.broadcasted_iota(jnp.int32, (1, n), 1).astype(
      jnp.float32) * float(ncells)
  cf = jnp.dot(cb_ref[...].astype(jnp.float32), w,
               preferred_element_type=jnp.float32)
  cf_ref[...] = (cf + offs).astype(jnp.int32)
  qf = jnp.dot(qb_ref[...].astype(jnp.float32), w,
               preferred_element_type=jnp.float32)
  qf_ref[...] = (qf + offs).astype(jnp.int32)


def _commit_body(caddr_hbm, vals_hbm, out_hbm,
                 abuf, vbuf, val_loc, s16, stage, mbuf, macc,
                 *, chunk, win, cells):
  c = lax.axis_index("c")
  s = lax.axis_index("s")
  wid = c * _NS + s
  base = wid * chunk
  nvec = win // _L
  nwin = chunk // win
  sl = cells // _NS

  iota = lax.iota(jnp.int32, _L)
  rot8 = (iota + 8) & (_L - 1)
  hi8 = iota >= (_L // 2)

  def init_body(i, _):
    val_loc[pl.ds(i * _L, _L)] = jnp.full((_L,), _SENT, jnp.float32)
    return 0
  lax.fori_loop(0, cells // _L, init_body, 0)

  def win_body(wi, _):
    wbase = base + wi * win
    pltpu.sync_copy(caddr_hbm.at[pl.ds(wbase, win)], abuf)
    pltpu.sync_copy(vals_hbm.at[pl.ds(wbase, win)], vbuf)

    def vec_body(i, _):
      flat = abuf[pl.ds(i * _L, _L)]
      val = vbuf[pl.ds(i * _L, _L)]
      s16[...] = flat
      other = plsc.load_gather(s16, [rot8])
      mask = hi8 | (flat != other)
      plsc.store_scatter(val_loc, [flat], val, mask=mask)
      return 0
    lax.fori_loop(0, nvec, vec_body, 0)
    return 0
  lax.fori_loop(0, nwin, win_body, 0)

  pltpu.sync_copy(val_loc, stage.at[s])
  plsc.subcore_barrier()
  sbase = s * sl
  pltpu.sync_copy(stage.at[0, pl.ds(sbase, sl)], macc)
  for t in range(1, _NS):
    pltpu.sync_copy(stage.at[t, pl.ds(sbase, sl)], mbuf)

    def mrg(i, _):
      v = mbuf[pl.ds(i * _L, _L)]
      a = macc[pl.ds(i * _L, _L)]
      macc[pl.ds(i * _L, _L)] = jnp.where(v >= 0.0, v, a)
      return 0
    lax.fori_loop(0, sl // _L, mrg, 0)
  pltpu.sync_copy(macc, out_hbm.at[c, pl.ds(sbase, sl)])


def _query_body(part_hbm, memflat_hbm, qaddr_hbm, out_hbm,
                t0, t1, tm, shmem, mem_loc, qbuf, obuf,
                *, chunk, win, cells):
  c = lax.axis_index("c")
  s = lax.axis_index("s")
  wid = c * _NS + s
  nvec = win // _L
  nwin = chunk // win
  sl = cells // _NS
  sbase = s * sl

  pltpu.sync_copy(part_hbm.at[0, pl.ds(sbase, sl)], t0)
  pltpu.sync_copy(part_hbm.at[1, pl.ds(sbase, sl)], t1)
  pltpu.sync_copy(memflat_hbm.at[pl.ds(sbase, sl)], tm)

  def mrg(i, _):
    v0 = t0[pl.ds(i * _L, _L)]
    v1 = t1[pl.ds(i * _L, _L)]
    m = tm[pl.ds(i * _L, _L)]
    m = jnp.where(v0 >= 0.0, v0, m)
    m = jnp.where(v1 >= 0.0, v1, m)
    tm[pl.ds(i * _L, _L)] = m
    return 0
  lax.fori_loop(0, sl // _L, mrg, 0)

  pltpu.sync_copy(tm, shmem.at[pl.ds(sbase, sl)])
  plsc.subcore_barrier()
  pltpu.sync_copy(shmem, mem_loc)

  base = wid * chunk

  def win_body(wi, _):
    wbase = base + wi * win
    pltpu.sync_copy(qaddr_hbm.at[pl.ds(wbase, win)], qbuf)

    def vec_body(i, _):
      q = qbuf[pl.ds(i * _L, _L)]
      obuf[pl.ds(i * _L, _L)] = plsc.load_gather(mem_loc, [q])
      return 0
    lax.fori_loop(0, nvec, vec_body, 0)
    pltpu.sync_copy(obuf, out_hbm.at[pl.ds(wbase, win)])
    return 0
  lax.fori_loop(0, nwin, win_body, 0)


def kernel(memory, commit_bits, commit_vals, query_bits, bit_sel):
  n, ncells = memory.shape
  b, ib = commit_bits.shape
  nb = bit_sel.shape[1]
  cells = n * ncells
  flat = b * n
  chunk = flat // _NW
  win = 4096
  blk = 8192

  pow2 = 2.0 ** jnp.arange(nb, dtype=jnp.float32)
  onehot = (bit_sel[:, :, None] ==
            jnp.arange(ib, dtype=bit_sel.dtype)[None, None, :])
  w = jnp.einsum("nki,k->in", onehot.astype(jnp.float32), pow2)

  caddr, qaddr = pl.pallas_call(
      functools.partial(_addr_body, ncells=ncells),
      grid=(b // blk,),
      in_specs=[
          pl.BlockSpec((blk, ib), lambda i: (i, 0)),
          pl.BlockSpec((blk, ib), lambda i: (i, 0)),
          pl.BlockSpec((ib, n), lambda i: (0, 0)),
      ],
      out_specs=[
          pl.BlockSpec((blk, n), lambda i: (i, 0)),
          pl.BlockSpec((blk, n), lambda i: (i, 0)),
      ],
      out_shape=[
          jax.ShapeDtypeStruct((b, n), jnp.int32),
          jax.ShapeDtypeStruct((b, n), jnp.int32),
      ],
  )(commit_bits, query_bits, w)

  mesh = plsc.VectorSubcoreMesh(core_axis_name="c", subcore_axis_name="s",
                                num_cores=_NC, num_subcores=_NS)

  commit_call = pl.kernel(
      functools.partial(_commit_body, chunk=chunk, win=win, cells=cells),
      out_type=jax.ShapeDtypeStruct((_NC, cells), jnp.float32),
      mesh=mesh,
      compiler_params=pltpu.CompilerParams(needs_layout_passes=False),
      scratch_types=[
          pltpu.VMEM((win,), jnp.int32),
          pltpu.VMEM((win,), jnp.float32),
          pltpu.VMEM((cells,), jnp.float32),
          pltpu.VMEM((_L,), jnp.int32),
          pltpu.VMEM_SHARED((_NS, cells), jnp.float32),
          pltpu.VMEM((cells // _NS,), jnp.float32),
          pltpu.VMEM((cells // _NS,), jnp.float32),
      ],
  )
  part = commit_call(caddr.reshape(-1), commit_vals.reshape(-1))

  query_call = pl.kernel(
      functools.partial(_query_body, chunk=chunk, win=win, cells=cells),
      out_type=jax.ShapeDtypeStruct((flat,), jnp.float32),
      mesh=mesh,
      compiler_params=pltpu.CompilerParams(needs_layout_passes=False),
      scratch_types=[
          pltpu.VMEM((cells // _NS,), jnp.float32),
          pltpu.VMEM((cells // _NS,), jnp.float32),
          pltpu.VMEM((cells // _NS,), jnp.float32),
          pltpu.VMEM_SHARED((cells,), jnp.float32),
          pltpu.VMEM((cells,), jnp.float32),
          pltpu.VMEM((win,), jnp.int32),
          pltpu.VMEM((win,), jnp.float32),
      ],
  )
  out = query_call(part, memory.reshape(-1), qaddr.reshape(-1))
  return out.reshape(b, n)


# speedup vs baseline: 177.0427x; 4.3962x over previous
import functools

import jax
import jax.numpy as jnp
from jax import lax
from jax.experimental import pallas as pl
from jax.experimental.pallas import tpu as pltpu
from jax.experimental.pallas import tpu_sc as plsc

_NC = 2
_NS = 16
_NW = _NC * _NS
_L = 16

_SENT = -1.0


def _addr_body(cbt_ref, qbt_ref, wt_ref, cf_ref, qf_ref):
  wt = wt_ref[...]
  cf = jnp.dot(wt, cbt_ref[...].astype(jnp.float32),
               preferred_element_type=jnp.float32)
  cf_ref[...] = cf.astype(jnp.int32)
  qf = jnp.dot(wt, qbt_ref[...].astype(jnp.float32),
               preferred_element_type=jnp.float32)
  qf_ref[...] = qf.astype(jnp.int32)


def _commit_body(caddr_hbm, vals_hbm, out_hbm,
                 abuf, vbuf, val_loc, stage, mbuf, macc,
                 *, chunk, win, ncells, tpn):
  c = lax.axis_index("c")
  s = lax.axis_index("s")
  wid = c * _NS + s
  neuron = wid // tpn
  quarter = wid % tpn
  nvec = win // _L
  nwin = chunk // win

  def init_body(i, _):
    val_loc[pl.ds(i * _L, _L)] = jnp.full((_L,), _SENT, jnp.float32)
    return 0
  lax.fori_loop(0, ncells // _L, init_body, 0)

  def win_body(wi, _):
    wbase = quarter * chunk + wi * win
    pltpu.sync_copy(caddr_hbm.at[neuron, pl.ds(wbase, win)], abuf)
    pltpu.sync_copy(vals_hbm.at[neuron, pl.ds(wbase, win)], vbuf)

    def vec_body(i, _):
      addr = abuf[pl.ds(i * _L, _L)]
      val = vbuf[pl.ds(i * _L, _L)]
      _, last = plsc.scan_count(addr)
      plsc.store_scatter(val_loc, [addr], val, mask=last)
      return 0
    lax.fori_loop(0, nvec, vec_body, 0)
    return 0
  lax.fori_loop(0, nwin, win_body, 0)

  pltpu.sync_copy(val_loc, stage.at[s])
  plsc.subcore_barrier()
  npersc = _NS // tpn
  nloc = s // tpn
  piece = s % tpn
  psize = ncells // tpn
  pbase = piece * psize
  pltpu.sync_copy(stage.at[nloc * tpn, pl.ds(pbase, psize)], macc)
  for t in range(1, tpn):
    pltpu.sync_copy(stage.at[nloc * tpn + t, pl.ds(pbase, psize)], mbuf)

    def mrg(i, _):
      v = mbuf[pl.ds(i * _L, _L)]
      a = macc[pl.ds(i * _L, _L)]
      macc[pl.ds(i * _L, _L)] = jnp.where(v >= 0.0, v, a)
      return 0
    lax.fori_loop(0, psize // _L, mrg, 0)
  pltpu.sync_copy(macc, out_hbm.at[c * npersc + nloc, pl.ds(pbase, psize)])


def _query_body(part_hbm, mem_hbm, qaddr_hbm, out_hbm,
                tpart, mem_loc, qbuf, obuf,
                *, chunk, win, ncells, tpn):
  c = lax.axis_index("c")
  s = lax.axis_index("s")
  wid = c * _NS + s
  neuron = wid // tpn
  quarter = wid % tpn
  nvec = win // _L
  nwin = chunk // win

  pltpu.sync_copy(part_hbm.at[neuron], tpart)
  pltpu.sync_copy(mem_hbm.at[neuron], mem_loc)

  def mrg(i, _):
    v = tpart[pl.ds(i * _L, _L)]
    m = mem_loc[pl.ds(i * _L, _L)]
    mem_loc[pl.ds(i * _L, _L)] = jnp.where(v >= 0.0, v, m)
    return 0
  lax.fori_loop(0, ncells // _L, mrg, 0)

  def win_body(wi, _):
    wbase = quarter * chunk + wi * win
    pltpu.sync_copy(qaddr_hbm.at[neuron, pl.ds(wbase, win)], qbuf)

    def vec_body(i, _):
      q = qbuf[pl.ds(i * _L, _L)]
      obuf[pl.ds(i * _L, _L)] = plsc.load_gather(mem_loc, [q])
      return 0
    lax.fori_loop(0, nvec, vec_body, 0)
    pltpu.sync_copy(obuf, out_hbm.at[neuron, pl.ds(wbase, win)])
    return 0
  lax.fori_loop(0, nwin, win_body, 0)


def kernel(memory, commit_bits, commit_vals, query_bits, bit_sel):
  n, ncells = memory.shape
  b, ib = commit_bits.shape
  nb = bit_sel.shape[1]
  tpn = _NW // n
  chunk = b // tpn
  win = 4096
  blk = 16384

  cbt = commit_bits.T
  qbt = query_bits.T
  valt = commit_vals.T

  pow2 = 2.0 ** jnp.arange(nb, dtype=jnp.float32)
  onehot = (bit_sel[:, :, None] ==
            jnp.arange(ib, dtype=bit_sel.dtype)[None, None, :])
  wt = jnp.einsum("nki,k->ni", onehot.astype(jnp.float32), pow2)

  caddr, qaddr = pl.pallas_call(
      _addr_body,
      grid=(b // blk,),
      in_specs=[
          pl.BlockSpec((ib, blk), lambda i: (0, i)),
          pl.BlockSpec((ib, blk), lambda i: (0, i)),
          pl.BlockSpec((n, ib), lambda i: (0, 0)),
      ],
      out_specs=[
          pl.BlockSpec((n, blk), lambda i: (0, i)),
          pl.BlockSpec((n, blk), lambda i: (0, i)),
      ],
      out_shape=[
          jax.ShapeDtypeStruct((n, b), jnp.int32),
          jax.ShapeDtypeStruct((n, b), jnp.int32),
      ],
  )(cbt, qbt, wt)

  mesh = plsc.VectorSubcoreMesh(core_axis_name="c", subcore_axis_name="s",
                                num_cores=_NC, num_subcores=_NS)

  commit_call = pl.kernel(
      functools.partial(_commit_body, chunk=chunk, win=win, ncells=ncells,
                        tpn=tpn),
      out_type=jax.ShapeDtypeStruct((n, ncells), jnp.float32),
      mesh=mesh,
      compiler_params=pltpu.CompilerParams(needs_layout_passes=False),
      scratch_types=[
          pltpu.VMEM((win,), jnp.int32),
          pltpu.VMEM((win,), jnp.float32),
          pltpu.VMEM((ncells,), jnp.float32),
          pltpu.VMEM_SHARED((_NS, ncells), jnp.float32),
          pltpu.VMEM((ncells // 4,), jnp.float32),
          pltpu.VMEM((ncells // 4,), jnp.float32),
      ],
  )
  part = commit_call(caddr, valt)

  query_call = pl.kernel(
      functools.partial(_query_body, chunk=chunk, win=win, ncells=ncells,
                        tpn=tpn),
      out_type=jax.ShapeDtypeStruct((n, b), jnp.float32),
      mesh=mesh,
      compiler_params=pltpu.CompilerParams(needs_layout_passes=False),
      scratch_types=[
          pltpu.VMEM((ncells,), jnp.float32),
          pltpu.VMEM((ncells,), jnp.float32),
          pltpu.VMEM((win,), jnp.int32),
          pltpu.VMEM((win,), jnp.float32),
      ],
  )
  out = query_call(part, memory, qaddr)
  return out.T


# speedup vs baseline: 282.8297x; 1.5975x over previous
import functools

import jax
import jax.numpy as jnp
from jax import lax
from jax.experimental import pallas as pl
from jax.experimental.pallas import tpu as pltpu
from jax.experimental.pallas import tpu_sc as plsc

_NC = 2
_NS = 16
_NW = _NC * _NS
_L = 16

_SENT = -1.0


def _addr_body(cbt_ref, qbt_ref, wt_ref, cf_ref, qf_ref):
  wt = wt_ref[...]
  cf = jnp.dot(wt, cbt_ref[...].astype(jnp.float32),
               preferred_element_type=jnp.float32)
  cf_ref[...] = cf.astype(jnp.int32)
  qf = jnp.dot(wt, qbt_ref[...].astype(jnp.float32),
               preferred_element_type=jnp.float32)
  qf_ref[...] = qf.astype(jnp.int32)


def _commit_body(caddr_hbm, vals_hbm, out_hbm,
                 abuf, vbuf, val_loc, stage, mbuf, macc,
                 *, chunk, win, ncells, tpn):
  c = lax.axis_index("c")
  s = lax.axis_index("s")
  wid = c * _NS + s
  neuron = wid // tpn
  quarter = wid % tpn
  nvec = win // _L
  nwin = chunk // win

  def init_body(i, _):
    val_loc[pl.ds(i * _L, _L)] = jnp.full((_L,), _SENT, jnp.float32)
    return 0
  lax.fori_loop(0, ncells // _L, init_body, 0)

  def win_body(wi, _):
    wbase = quarter * chunk + wi * win
    pltpu.sync_copy(caddr_hbm.at[neuron, pl.ds(wbase, win)], abuf)
    pltpu.sync_copy(vals_hbm.at[neuron, pl.ds(wbase, win)], vbuf)

    def vec_body(i, _):
      addrs = []
      vals = []
      for u in range(4):
        addrs.append(abuf[pl.ds((i * 4 + u) * _L, _L)])
        vals.append(vbuf[pl.ds((i * 4 + u) * _L, _L)])
      lasts = [plsc.scan_count(a)[1] for a in addrs]
      for u in range(4):
        plsc.store_scatter(val_loc, [addrs[u]], vals[u], mask=lasts[u])
      return 0
    lax.fori_loop(0, nvec // 4, vec_body, 0)
    return 0
  lax.fori_loop(0, nwin, win_body, 0)

  pltpu.sync_copy(val_loc, stage.at[s])
  plsc.subcore_barrier()
  npersc = _NS // tpn
  nloc = s // tpn
  piece = s % tpn
  psize = ncells // tpn
  pbase = piece * psize
  pltpu.sync_copy(stage.at[nloc * tpn, pl.ds(pbase, psize)], macc)
  for t in range(1, tpn):
    pltpu.sync_copy(stage.at[nloc * tpn + t, pl.ds(pbase, psize)], mbuf)

    def mrg(i, _):
      v = mbuf[pl.ds(i * _L, _L)]
      a = macc[pl.ds(i * _L, _L)]
      macc[pl.ds(i * _L, _L)] = jnp.where(v >= 0.0, v, a)
      return 0
    lax.fori_loop(0, psize // _L, mrg, 0)
  pltpu.sync_copy(macc, out_hbm.at[c * npersc + nloc, pl.ds(pbase, psize)])


def _query_body(part_hbm, mem_hbm, qaddr_hbm, out_hbm,
                tpart, mem_loc, qbuf, obuf,
                *, chunk, win, ncells, tpn):
  c = lax.axis_index("c")
  s = lax.axis_index("s")
  wid = c * _NS + s
  neuron = wid // tpn
  quarter = wid % tpn
  nvec = win // _L
  nwin = chunk // win

  pltpu.sync_copy(part_hbm.at[neuron], tpart)
  pltpu.sync_copy(mem_hbm.at[neuron], mem_loc)

  def mrg(i, _):
    v = tpart[pl.ds(i * _L, _L)]
    m = mem_loc[pl.ds(i * _L, _L)]
    mem_loc[pl.ds(i * _L, _L)] = jnp.where(v >= 0.0, v, m)
    return 0
  lax.fori_loop(0, ncells // _L, mrg, 0)

  def win_body(wi, _):
    wbase = quarter * chunk + wi * win
    pltpu.sync_copy(qaddr_hbm.at[neuron, pl.ds(wbase, win)], qbuf)

    def vec_body(i, _):
      qs = [qbuf[pl.ds((i * 4 + u) * _L, _L)] for u in range(4)]
      rs = [plsc.load_gather(mem_loc, [q]) for q in qs]
      for u in range(4):
        obuf[pl.ds((i * 4 + u) * _L, _L)] = rs[u]
      return 0
    lax.fori_loop(0, nvec // 4, vec_body, 0)
    pltpu.sync_copy(obuf, out_hbm.at[neuron, pl.ds(wbase, win)])
    return 0
  lax.fori_loop(0, nwin, win_body, 0)


def kernel(memory, commit_bits, commit_vals, query_bits, bit_sel):
  n, ncells = memory.shape
  b, ib = commit_bits.shape
  nb = bit_sel.shape[1]
  tpn = _NW // n
  chunk = b // tpn
  win = 4096
  blk = 16384

  cbt = commit_bits.T
  qbt = query_bits.T
  valt = commit_vals.T

  pow2 = 2.0 ** jnp.arange(nb, dtype=jnp.float32)
  onehot = (bit_sel[:, :, None] ==
            jnp.arange(ib, dtype=bit_sel.dtype)[None, None, :])
  wt = jnp.einsum("nki,k->ni", onehot.astype(jnp.float32), pow2)

  caddr, qaddr = pl.pallas_call(
      _addr_body,
      grid=(b // blk,),
      in_specs=[
          pl.BlockSpec((ib, blk), lambda i: (0, i)),
          pl.BlockSpec((ib, blk), lambda i: (0, i)),
          pl.BlockSpec((n, ib), lambda i: (0, 0)),
      ],
      out_specs=[
          pl.BlockSpec((n, blk), lambda i: (0, i)),
          pl.BlockSpec((n, blk), lambda i: (0, i)),
      ],
      out_shape=[
          jax.ShapeDtypeStruct((n, b), jnp.int32),
          jax.ShapeDtypeStruct((n, b), jnp.int32),
      ],
  )(cbt, qbt, wt)

  mesh = plsc.VectorSubcoreMesh(core_axis_name="c", subcore_axis_name="s",
                                num_cores=_NC, num_subcores=_NS)

  commit_call = pl.kernel(
      functools.partial(_commit_body, chunk=chunk, win=win, ncells=ncells,
                        tpn=tpn),
      out_type=jax.ShapeDtypeStruct((n, ncells), jnp.float32),
      mesh=mesh,
      compiler_params=pltpu.CompilerParams(needs_layout_passes=False),
      scratch_types=[
          pltpu.VMEM((win,), jnp.int32),
          pltpu.VMEM((win,), jnp.float32),
          pltpu.VMEM((ncells,), jnp.float32),
          pltpu.VMEM_SHARED((_NS, ncells), jnp.float32),
          pltpu.VMEM((ncells // 4,), jnp.float32),
          pltpu.VMEM((ncells // 4,), jnp.float32),
      ],
  )
  part = commit_call(caddr, valt)

  query_call = pl.kernel(
      functools.partial(_query_body, chunk=chunk, win=win, ncells=ncells,
                        tpn=tpn),
      out_type=jax.ShapeDtypeStruct((n, b), jnp.float32),
      mesh=mesh,
      compiler_params=pltpu.CompilerParams(needs_layout_passes=False),
      scratch_types=[
          pltpu.VMEM((ncells,), jnp.float32),
          pltpu.VMEM((ncells,), jnp.float32),
          pltpu.VMEM((win,), jnp.int32),
          pltpu.VMEM((win,), jnp.float32),
      ],
  )
  out = query_call(part, memory, qaddr)
  return out.T


# speedup vs baseline: 422.9173x; 1.4953x over previous
import functools

import jax
import jax.numpy as jnp
from jax import lax
from jax.experimental import pallas as pl
from jax.experimental.pallas import tpu as pltpu
from jax.experimental.pallas import tpu_sc as plsc

_NC = 2
_NS = 16
_NW = _NC * _NS
_L = 16

_SENT = -1.0


def _addr_body(cbt_ref, qbt_ref, wt_ref, cf_ref, qf_ref):
  wt = wt_ref[...]
  cf = jnp.dot(wt, cbt_ref[...].astype(jnp.float32),
               preferred_element_type=jnp.float32)
  cf_ref[...] = cf.astype(jnp.int32)
  qf = jnp.dot(wt, qbt_ref[...].astype(jnp.float32),
               preferred_element_type=jnp.float32)
  qf_ref[...] = qf.astype(jnp.int32)


def _sc_body(caddr_hbm, vals_hbm, mem_hbm, qaddr_hbm, out_hbm,
             abuf0, vbuf0, abuf1, vbuf1, val_loc, stage, tbuf, mem_loc,
             qbuf0, qbuf1, obuf0, obuf1,
             sem0, sem1, semq0, semq1, semo0, semo1,
             *, chunk, win, ncells, tpn):
  c = lax.axis_index("c")
  s = lax.axis_index("s")
  wid = c * _NS + s
  neuron = wid // tpn
  quarter = wid % tpn
  nvec = win // _L
  nwin = chunk // win
  base = quarter * chunk

  def init_body(i, _):
    val_loc[pl.ds(i * _L, _L)] = jnp.full((_L,), _SENT, jnp.float32)
    return 0
  lax.fori_loop(0, ncells // _L, init_body, 0)

  def cfetch(w, ab, vb, sem):
    pltpu.async_copy(caddr_hbm.at[neuron, pl.ds(base + w * win, win)], ab, sem)
    pltpu.async_copy(vals_hbm.at[neuron, pl.ds(base + w * win, win)], vb, sem)

  def cwait(w, ab, vb, sem):
    pltpu.make_async_copy(
        caddr_hbm.at[neuron, pl.ds(base + w * win, win)], ab, sem).wait()
    pltpu.make_async_copy(
        vals_hbm.at[neuron, pl.ds(base + w * win, win)], vb, sem).wait()

  def commit_window(ab, vb):
    def vec_body(i, _):
      addrs = [ab[pl.ds((i * 4 + u) * _L, _L)] for u in range(4)]
      vals = [vb[pl.ds((i * 4 + u) * _L, _L)] for u in range(4)]
      lasts = [plsc.scan_count(a)[1] for a in addrs]
      for u in range(4):
        plsc.store_scatter(val_loc, [addrs[u]], vals[u], mask=lasts[u])
      return 0
    lax.fori_loop(0, nvec // 4, vec_body, 0)

  cfetch(0, abuf0, vbuf0, sem0)

  def cbody(g, _):
    w0 = 2 * g
    w1 = 2 * g + 1
    wn = jnp.minimum(2 * g + 2, nwin - 1)
    cwait(w0, abuf0, vbuf0, sem0)
    cfetch(w1, abuf1, vbuf1, sem1)
    commit_window(abuf0, vbuf0)
    cwait(w1, abuf1, vbuf1, sem1)
    cfetch(wn, abuf0, vbuf0, sem0)
    commit_window(abuf1, vbuf1)
    return 0
  lax.fori_loop(0, nwin // 2, cbody, 0)
  cwait(nwin - 1, abuf0, vbuf0, sem0)

  pltpu.sync_copy(val_loc, stage.at[s])
  plsc.subcore_barrier()
  pltpu.sync_copy(mem_hbm.at[neuron], mem_loc)
  grp = (s // tpn) * tpn
  for t in range(tpn):
    pltpu.sync_copy(stage.at[grp + t], tbuf)

    def mrg(i, _):
      for u in range(4):
        v = tbuf[pl.ds((i * 4 + u) * _L, _L)]
        m = mem_loc[pl.ds((i * 4 + u) * _L, _L)]
        mem_loc[pl.ds((i * 4 + u) * _L, _L)] = jnp.where(v >= 0.0, v, m)
      return 0
    lax.fori_loop(0, ncells // _L // 4, mrg, 0)

  def qfetch(w, qb, sem):
    pltpu.async_copy(qaddr_hbm.at[neuron, pl.ds(base + w * win, win)], qb, sem)

  def qwait(w, qb, sem):
    pltpu.make_async_copy(
        qaddr_hbm.at[neuron, pl.ds(base + w * win, win)], qb, sem).wait()

  def ostart(w, ob, sem):
    pltpu.async_copy(ob, out_hbm.at[neuron, pl.ds(base + w * win, win)], sem)

  def owait(w, ob, sem):
    pltpu.make_async_copy(
        ob, out_hbm.at[neuron, pl.ds(base + w * win, win)], sem).wait()

  def query_window(qb, ob):
    def vec_body(i, _):
      qs = [qb[pl.ds((i * 4 + u) * _L, _L)] for u in range(4)]
      rs = [plsc.load_gather(mem_loc, [q]) for q in qs]
      for u in range(4):
        ob[pl.ds((i * 4 + u) * _L, _L)] = rs[u]
      return 0
    lax.fori_loop(0, nvec // 4, vec_body, 0)

  qfetch(0, qbuf0, semq0)

  def qbody(g, _):
    w0 = 2 * g
    w1 = 2 * g + 1
    wn = jnp.minimum(2 * g + 2, nwin - 1)
    qwait(w0, qbuf0, semq0)
    qfetch(w1, qbuf1, semq1)

    @pl.when(g > 0)
    def _():
      owait(w0 - 2, obuf0, semo0)
    query_window(qbuf0, obuf0)
    ostart(w0, obuf0, semo0)

    qwait(w1, qbuf1, semq1)
    qfetch(wn, qbuf0, semq0)

    @pl.when(g > 0)
    def _():
      owait(w1 - 2, obuf1, semo1)
    query_window(qbuf1, obuf1)
    ostart(w1, obuf1, semo1)
    return 0
  lax.fori_loop(0, nwin // 2, qbody, 0)
  qwait(nwin - 1, qbuf0, semq0)
  owait(nwin - 2, obuf0, semo0)
  owait(nwin - 1, obuf1, semo1)


def kernel(memory, commit_bits, commit_vals, query_bits, bit_sel):
  n, ncells = memory.shape
  b, ib = commit_bits.shape
  nb = bit_sel.shape[1]
  tpn = _NW // n
  chunk = b // tpn
  win = 4096
  blk = 16384

  cbt = commit_bits.T
  qbt = query_bits.T
  valt = commit_vals.T

  pow2 = 2.0 ** jnp.arange(nb, dtype=jnp.float32)
  onehot = (bit_sel[:, :, None] ==
            jnp.arange(ib, dtype=bit_sel.dtype)[None, None, :])
  wt = jnp.einsum("nki,k->ni", onehot.astype(jnp.float32), pow2)

  caddr, qaddr = pl.pallas_call(
      _addr_body,
      grid=(b // blk,),
      in_specs=[
          pl.BlockSpec((ib, blk), lambda i: (0, i)),
          pl.BlockSpec((ib, blk), lambda i: (0, i)),
          pl.BlockSpec((n, ib), lambda i: (0, 0)),
      ],
      out_specs=[
          pl.BlockSpec((n, blk), lambda i: (0, i)),
          pl.BlockSpec((n, blk), lambda i: (0, i)),
      ],
      out_shape=[
          jax.ShapeDtypeStruct((n, b), jnp.int32),
          jax.ShapeDtypeStruct((n, b), jnp.int32),
      ],
  )(cbt, qbt, wt)

  mesh = plsc.VectorSubcoreMesh(core_axis_name="c", subcore_axis_name="s",
                                num_cores=_NC, num_subcores=_NS)

  sc_call = pl.kernel(
      functools.partial(_sc_body, chunk=chunk, win=win, ncells=ncells,
                        tpn=tpn),
      out_type=jax.ShapeDtypeStruct((n, b), jnp.float32),
      mesh=mesh,
      compiler_params=pltpu.CompilerParams(needs_layout_passes=False),
      scratch_types=[
          pltpu.VMEM((win,), jnp.int32),
          pltpu.VMEM((win,), jnp.float32),
          pltpu.VMEM((win,), jnp.int32),
          pltpu.VMEM((win,), jnp.float32),
          pltpu.VMEM((ncells,), jnp.float32),
          pltpu.VMEM_SHARED((_NS, ncells), jnp.float32),
          pltpu.VMEM((ncells,), jnp.float32),
          pltpu.VMEM((ncells,), jnp.float32),
          pltpu.VMEM((win,), jnp.int32),
          pltpu.VMEM((win,), jnp.int32),
          pltpu.VMEM((win,), jnp.float32),
          pltpu.VMEM((win,), jnp.float32),
          pltpu.SemaphoreType.DMA,
          pltpu.SemaphoreType.DMA,
          pltpu.SemaphoreType.DMA,
          pltpu.SemaphoreType.DMA,
          pltpu.SemaphoreType.DMA,
          pltpu.SemaphoreType.DMA,
      ],
  )
  out = sc_call(caddr, valt, memory, qaddr)
  return out.T
